# baseline (speedup 1.0000x reference)
import jax
import jax.numpy as jnp
from jax import lax
from jax.experimental import pallas as pl
from jax.experimental.pallas import tpu as pltpu
from jax.experimental.pallas import tpu_sc as plsc

NUM_EMBEDDINGS = 1000000
EMBEDDING_DIM = 64
BATCH = 16384
HIST = 20

TOTAL = BATCH * HIST
NUM_CORES = 2
NUM_SUBCORES = 16
NUM_WORKERS = NUM_CORES * NUM_SUBCORES
PER_WORKER = TOTAL // NUM_WORKERS
CHUNK = 512
NUM_CHUNKS = PER_WORKER // CHUNK
NBUF = 3

LANES = 128
FULL_TC = NUM_EMBEDDINGS // LANES
TAIL = NUM_EMBEDDINGS - FULL_TC * LANES
TC_LO = FULL_TC // NUM_WORKERS
TC_EXTRA = FULL_TC - TC_LO * NUM_WORKERS

assert TOTAL % NUM_WORKERS == 0
assert PER_WORKER % CHUNK == 0


def _relayout_body(wt_hbm, tail_hbm, out_hbm, in_bufs, out_bufs, isems, osems):
    wid = lax.axis_index("s") * NUM_CORES + lax.axis_index("c")
    base_tc = wid * TC_LO + jnp.minimum(wid, TC_EXTRA)
    n_tc = TC_LO + jnp.where(wid < TC_EXTRA, 1, 0)

    lane_iota = lax.iota(jnp.int32, 16)

    def start_load(tc, b):
        return pltpu.async_copy(
            wt_hbm.at[:, pl.ds(tc * LANES, LANES)], in_bufs.at[b], isems[b])

    def transpose_block(b, nlanes):
        def col(vl, carry):
            for h in range(EMBEDDING_DIM // 16):
                rows = lane_iota + (16 * h)
                cols = jnp.full((16,), vl, jnp.int32)
                g = plsc.load_gather(in_bufs.at[b], [rows, cols])
                out_bufs[b, pl.ds(vl * EMBEDDING_DIM + 16 * h, 16)] = g
            return carry
        lax.fori_loop(0, nlanes, col, 0)

    def loop1(i, carry):
        tc = base_tc + i
        start_load(tc, 0).wait()
        transpose_block(0, LANES)
        pltpu.async_copy(
            out_bufs.at[0],
            out_hbm.at[pl.ds(tc * LANES * EMBEDDING_DIM,
                             LANES * EMBEDDING_DIM)],
            osems[0]).wait()
        return carry

    lax.fori_loop(0, n_tc, loop1, 0)

    @pl.when(wid == NUM_WORKERS - 1)
    def _tail():
        pltpu.async_copy(
            tail_hbm, out_bufs.at[1, pl.ds(0, TAIL * EMBEDDING_DIM)],
            isems[1]).wait()
        pltpu.async_copy(
            out_bufs.at[1, pl.ds(0, TAIL * EMBEDDING_DIM)],
            out_hbm.at[pl.ds(FULL_TC * LANES * EMBEDDING_DIM,
                             TAIL * EMBEDDING_DIM)],
            osems[1]).wait()


def _gather_body(x_hbm, w_hbm, out_hbm, idx_v, rows_v, gsems, ssems):
    wid = lax.axis_index("s") * NUM_CORES + lax.axis_index("c")
    base = wid * PER_WORKER

    pltpu.sync_copy(x_hbm.at[pl.ds(base, PER_WORKER)], idx_v)

    def start_gather(c):
        b = c % NBUF
        return pltpu.async_copy(
            w_hbm.at[idx_v.at[pl.ds(c * CHUNK, CHUNK)]], rows_v.at[b],
            gsems[b])

    gd = {}
    for c in range(NBUF):
        gd[c] = start_gather(c)

    pending_stores = {}
    for c in range(NUM_CHUNKS):
        b = c % NBUF
        gd[c].wait()
        sd = pltpu.async_copy(
            rows_v.at[b], out_hbm.at[pl.ds(base + c * CHUNK, CHUNK)],
            ssems[b])
        if c + NBUF < NUM_CHUNKS:
            sd.wait()
            gd[c + NBUF] = start_gather(c + NBUF)
        else:
            pending_stores[b] = sd

    for b in sorted(pending_stores):
        pending_stores[b].wait()


@jax.jit
def kernel(x, weight):
    mesh = plsc.VectorSubcoreMesh(
        core_axis_name="c", subcore_axis_name="s",
        num_cores=NUM_CORES, num_subcores=NUM_SUBCORES,
    )
    dense = pl.kernel(
        _relayout_body,
        out_type=jax.ShapeDtypeStruct((NUM_EMBEDDINGS * EMBEDDING_DIM,),
                                      jnp.float32),
        mesh=mesh,
        scratch_types=[
            pltpu.VMEM((2, EMBEDDING_DIM, LANES), jnp.float32),
            pltpu.VMEM((2, LANES * EMBEDDING_DIM), jnp.float32),
            [pltpu.SemaphoreType.DMA] * 2,
            [pltpu.SemaphoreType.DMA] * 2,
        ],
        compiler_params=pltpu.CompilerParams(use_tc_tiling_on_sc=True,
                                             needs_layout_passes=False),
    )(weight.T,
      lax.slice(weight, (FULL_TC * LANES, 0),
                (NUM_EMBEDDINGS, EMBEDDING_DIM)).reshape(TAIL * EMBEDDING_DIM))
    table = dense.reshape(NUM_EMBEDDINGS, EMBEDDING_DIM)

    flat_x = x.reshape(TOTAL)
    out = pl.kernel(
        _gather_body,
        out_type=jax.ShapeDtypeStruct((TOTAL, EMBEDDING_DIM), jnp.float32),
        mesh=mesh,
        scratch_types=[
            pltpu.VMEM((PER_WORKER,), jnp.int32),
            pltpu.VMEM((NBUF, CHUNK, EMBEDDING_DIM), jnp.float32),
            [pltpu.SemaphoreType.DMA] * NBUF,
            [pltpu.SemaphoreType.DMA] * NBUF,
        ],
        compiler_params=pltpu.CompilerParams(use_tc_tiling_on_sc=False),
    )(flat_x, table)
    return out.reshape(BATCH, HIST, EMBEDDING_DIM)


# speedup vs baseline: 1.3906x; 1.3906x over previous
import jax
import jax.numpy as jnp
from jax import lax
from jax.experimental import pallas as pl
from jax.experimental.pallas import tpu as pltpu
from jax.experimental.pallas import tpu_sc as plsc

NUM_EMBEDDINGS = 1000000
EMBEDDING_DIM = 64
BATCH = 16384
HIST = 20

TOTAL = BATCH * HIST
NUM_CORES = 2
NUM_SUBCORES = 16
NUM_WORKERS = NUM_CORES * NUM_SUBCORES
PER_WORKER = TOTAL // NUM_WORKERS
CHUNK = 512
NUM_CHUNKS = PER_WORKER // CHUNK
NBUF = 3

LANES = 128
FULL_TC = NUM_EMBEDDINGS // LANES
TAIL = NUM_EMBEDDINGS - FULL_TC * LANES
TC_LO = FULL_TC // NUM_WORKERS
TC_EXTRA = FULL_TC - TC_LO * NUM_WORKERS

assert TOTAL % NUM_WORKERS == 0
assert PER_WORKER % CHUNK == 0


def _relayout_body(wt_hbm, tail_hbm, out_hbm, in0, in1, ob0, ob1,
                   isems, osems):
    in_bufs = (in0, in1)
    out_bufs = (ob0, ob1)
    wid = lax.axis_index("s") * NUM_CORES + lax.axis_index("c")
    base_tc = wid * TC_LO + jnp.minimum(wid, TC_EXTRA)
    n_tc = TC_LO + jnp.where(wid < TC_EXTRA, 1, 0)

    lane_iota = lax.iota(jnp.int32, 16)
    bases = [lane_iota * EMBEDDING_DIM + 16 * k * EMBEDDING_DIM
             for k in range(LANES // 16)]

    def start_load(tc, b):
        return pltpu.async_copy(
            wt_hbm.at[:, pl.ds(tc * LANES, LANES)], in_bufs[b], isems[b])

    def wait_load(b):
        pltpu.make_async_copy(
            wt_hbm.at[:, pl.ds(0, LANES)], in_bufs[b], isems[b]).wait()

    def start_store(tc, b):
        pltpu.async_copy(
            out_bufs[b],
            out_hbm.at[pl.ds(tc * LANES * EMBEDDING_DIM,
                             LANES * EMBEDDING_DIM)],
            osems[b])

    def wait_store(b):
        pltpu.make_async_copy(
            out_bufs[b],
            out_hbm.at[pl.ds(0, LANES * EMBEDDING_DIM)], osems[b]).wait()

    def transpose_block(b):
        for d in range(EMBEDDING_DIM):
            rows = jnp.full((16,), d, jnp.int32)
            for k in range(LANES // 16):
                v = plsc.load_gather(in_bufs[b], [rows, lane_iota + 16 * k])
                plsc.store_scatter(out_bufs[b], [bases[k] + d], v)

    start_load(base_tc, 0)
    start_load(base_tc + 1, 1)

    def pair(g, carry):
        for b in (0, 1):
            i = 2 * g + b

            @pl.when(i < n_tc)
            def _():
                tc = base_tc + i
                wait_load(b)

                @pl.when(i >= 2)
                def _():
                    wait_store(b)

                transpose_block(b)
                start_store(tc, b)

                @pl.when(i + 2 < n_tc)
                def _():
                    start_load(tc + 2, b)
        return carry

    lax.fori_loop(0, (TC_LO + 2) // 2, pair, 0)
    wait_store(0)
    wait_store(1)

    @pl.when(wid == NUM_WORKERS - 1)
    def _tail():
        pltpu.async_copy(
            tail_hbm, out_bufs[1].at[pl.ds(0, TAIL * EMBEDDING_DIM)],
            isems[1]).wait()
        pltpu.async_copy(
            out_bufs[1].at[pl.ds(0, TAIL * EMBEDDING_DIM)],
            out_hbm.at[pl.ds(FULL_TC * LANES * EMBEDDING_DIM,
                             TAIL * EMBEDDING_DIM)],
            osems[1]).wait()


def _gather_body(x_hbm, w_hbm, out_hbm, idx_v, rows_v, gsems, ssems):
    wid = lax.axis_index("s") * NUM_CORES + lax.axis_index("c")
    base = wid * PER_WORKER

    pltpu.sync_copy(x_hbm.at[pl.ds(base, PER_WORKER)], idx_v)

    def start_gather(c):
        b = c % NBUF
        return pltpu.async_copy(
            w_hbm.at[idx_v.at[pl.ds(c * CHUNK, CHUNK)]], rows_v.at[b],
            gsems[b])

    gd = {}
    for c in range(NBUF):
        gd[c] = start_gather(c)

    pending_stores = {}
    for c in range(NUM_CHUNKS):
        b = c % NBUF
        gd[c].wait()
        sd = pltpu.async_copy(
            rows_v.at[b], out_hbm.at[pl.ds(base + c * CHUNK, CHUNK)],
            ssems[b])
        if c + NBUF < NUM_CHUNKS:
            sd.wait()
            gd[c + NBUF] = start_gather(c + NBUF)
        else:
            pending_stores[b] = sd

    for b in sorted(pending_stores):
        pending_stores[b].wait()


@jax.jit
def kernel(x, weight):
    mesh = plsc.VectorSubcoreMesh(
        core_axis_name="c", subcore_axis_name="s",
        num_cores=NUM_CORES, num_subcores=NUM_SUBCORES,
    )
    dense = pl.kernel(
        _relayout_body,
        out_type=jax.ShapeDtypeStruct((NUM_EMBEDDINGS * EMBEDDING_DIM,),
                                      jnp.float32),
        mesh=mesh,
        scratch_types=[
            pltpu.VMEM((EMBEDDING_DIM, LANES), jnp.float32),
            pltpu.VMEM((EMBEDDING_DIM, LANES), jnp.float32),
            pltpu.VMEM((LANES * EMBEDDING_DIM,), jnp.float32),
            pltpu.VMEM((LANES * EMBEDDING_DIM,), jnp.float32),
            [pltpu.SemaphoreType.DMA] * 2,
            [pltpu.SemaphoreType.DMA] * 2,
        ],
        compiler_params=pltpu.CompilerParams(use_tc_tiling_on_sc=True,
                                             needs_layout_passes=False),
    )(weight.T,
      lax.slice(weight, (FULL_TC * LANES, 0),
                (NUM_EMBEDDINGS, EMBEDDING_DIM)).reshape(TAIL * EMBEDDING_DIM))
    table = dense.reshape(NUM_EMBEDDINGS, EMBEDDING_DIM)

    flat_x = x.reshape(TOTAL)
    out = pl.kernel(
        _gather_body,
        out_type=jax.ShapeDtypeStruct((TOTAL, EMBEDDING_DIM), jnp.float32),
        mesh=mesh,
        scratch_types=[
            pltpu.VMEM((PER_WORKER,), jnp.int32),
            pltpu.VMEM((NBUF, CHUNK, EMBEDDING_DIM), jnp.float32),
            [pltpu.SemaphoreType.DMA] * NBUF,
            [pltpu.SemaphoreType.DMA] * NBUF,
        ],
        compiler_params=pltpu.CompilerParams(use_tc_tiling_on_sc=False),
    )(flat_x, table)
    return out.reshape(BATCH, HIST, EMBEDDING_DIM)


# speedup vs baseline: 1.8522x; 1.3319x over previous
import jax
import jax.numpy as jnp
from jax import lax
from jax.experimental import pallas as pl
from jax.experimental.pallas import tpu as pltpu
from jax.experimental.pallas import tpu_sc as plsc

NUM_EMBEDDINGS = 1000000
EMBEDDING_DIM = 64
BATCH = 16384
HIST = 20

TOTAL = BATCH * HIST
NUM_CORES = 2
NUM_SUBCORES = 16
NUM_WORKERS = NUM_CORES * NUM_SUBCORES
PER_WORKER = TOTAL // NUM_WORKERS
CHUNK = 512
NUM_CHUNKS = PER_WORKER // CHUNK
NBUF = 3

LANES = 128
FULL_TC = NUM_EMBEDDINGS // LANES
TAIL = NUM_EMBEDDINGS - FULL_TC * LANES
TC_LO = FULL_TC // NUM_WORKERS
TC_EXTRA = FULL_TC - TC_LO * NUM_WORKERS

assert TOTAL % NUM_WORKERS == 0
assert PER_WORKER % CHUNK == 0


def _relayout_body(wt_hbm, tail_hbm, out_hbm, in0, in1, ob0, ob1,
                   isems, osems):
    in_bufs = (in0, in1)
    out_bufs = (ob0, ob1)
    wid = lax.axis_index("s") * NUM_CORES + lax.axis_index("c")
    base_tc = wid * TC_LO + jnp.minimum(wid, TC_EXTRA)
    n_tc = TC_LO + jnp.where(wid < TC_EXTRA, 1, 0)

    lane_iota = lax.iota(jnp.int32, 16)
    bases = [lane_iota * EMBEDDING_DIM + 16 * k * EMBEDDING_DIM
             for k in range(LANES // 16)]

    def start_load(tc, b):
        return pltpu.async_copy(
            wt_hbm.at[:, pl.ds(tc * LANES, LANES)], in_bufs[b], isems[b])

    def wait_load(b):
        pltpu.make_async_copy(
            wt_hbm.at[:, pl.ds(0, LANES)], in_bufs[b], isems[b]).wait()

    def start_store(tc, b):
        pltpu.async_copy(
            out_bufs[b],
            out_hbm.at[pl.ds(tc * LANES * EMBEDDING_DIM,
                             LANES * EMBEDDING_DIM)],
            osems[b])

    def wait_store(b):
        pltpu.make_async_copy(
            out_bufs[b],
            out_hbm.at[pl.ds(0, LANES * EMBEDDING_DIM)], osems[b]).wait()

    def transpose_block(b):
        inb = in_bufs[b]
        outb = out_bufs[b]

        @plsc.parallel_loop(0, EMBEDDING_DIM, step=1)
        def _(d):
            rows = jnp.full((16,), d, jnp.int32)
            for k in range(LANES // 16):
                v = plsc.load_gather(inb, [rows, lane_iota + 16 * k])
                plsc.store_scatter(outb, [bases[k] + d], v)

    start_load(base_tc, 0)
    start_load(base_tc + 1, 1)

    def pair(g, carry):
        for b in (0, 1):
            i = 2 * g + b

            @pl.when(i < n_tc)
            def _():
                tc = base_tc + i
                wait_load(b)

                @pl.when(i >= 2)
                def _():
                    wait_store(b)

                transpose_block(b)
                start_store(tc, b)

                @pl.when(i + 2 < n_tc)
                def _():
                    start_load(tc + 2, b)
        return carry

    lax.fori_loop(0, (TC_LO + 2) // 2, pair, 0)
    wait_store(0)
    wait_store(1)

    @pl.when(wid == NUM_WORKERS - 1)
    def _tail():
        pltpu.async_copy(
            tail_hbm, out_bufs[1].at[pl.ds(0, TAIL * EMBEDDING_DIM)],
            isems[1]).wait()
        pltpu.async_copy(
            out_bufs[1].at[pl.ds(0, TAIL * EMBEDDING_DIM)],
            out_hbm.at[pl.ds(FULL_TC * LANES * EMBEDDING_DIM,
                             TAIL * EMBEDDING_DIM)],
            osems[1]).wait()


def _gather_body(x_hbm, w_hbm, out_hbm, idx_v, rows_v, gsems, ssems):
    wid = lax.axis_index("s") * NUM_CORES + lax.axis_index("c")
    base = wid * PER_WORKER

    pltpu.sync_copy(x_hbm.at[pl.ds(base, PER_WORKER)], idx_v)

    def start_gather(c):
        b = c % NBUF
        return pltpu.async_copy(
            w_hbm.at[idx_v.at[pl.ds(c * CHUNK, CHUNK)]], rows_v.at[b],
            gsems[b])

    gd = {}
    for c in range(NBUF):
        gd[c] = start_gather(c)

    pending_stores = {}
    for c in range(NUM_CHUNKS):
        b = c % NBUF
        gd[c].wait()
        sd = pltpu.async_copy(
            rows_v.at[b], out_hbm.at[pl.ds(base + c * CHUNK, CHUNK)],
            ssems[b])
        if c + NBUF < NUM_CHUNKS:
            sd.wait()
            gd[c + NBUF] = start_gather(c + NBUF)
        else:
            pending_stores[b] = sd

    for b in sorted(pending_stores):
        pending_stores[b].wait()


@jax.jit
def kernel(x, weight):
    mesh = plsc.VectorSubcoreMesh(
        core_axis_name="c", subcore_axis_name="s",
        num_cores=NUM_CORES, num_subcores=NUM_SUBCORES,
    )
    dense = pl.kernel(
        _relayout_body,
        out_type=jax.ShapeDtypeStruct((NUM_EMBEDDINGS * EMBEDDING_DIM,),
                                      jnp.float32),
        mesh=mesh,
        scratch_types=[
            pltpu.VMEM((EMBEDDING_DIM, LANES), jnp.float32),
            pltpu.VMEM((EMBEDDING_DIM, LANES), jnp.float32),
            pltpu.VMEM((LANES * EMBEDDING_DIM,), jnp.float32),
            pltpu.VMEM((LANES * EMBEDDING_DIM,), jnp.float32),
            [pltpu.SemaphoreType.DMA] * 2,
            [pltpu.SemaphoreType.DMA] * 2,
        ],
        compiler_params=pltpu.CompilerParams(use_tc_tiling_on_sc=True,
                                             needs_layout_passes=False),
    )(weight.T,
      lax.slice(weight, (FULL_TC * LANES, 0),
                (NUM_EMBEDDINGS, EMBEDDING_DIM)).reshape(TAIL * EMBEDDING_DIM))
    table = dense.reshape(NUM_EMBEDDINGS, EMBEDDING_DIM)

    flat_x = x.reshape(TOTAL)
    out = pl.kernel(
        _gather_body,
        out_type=jax.ShapeDtypeStruct((TOTAL, EMBEDDING_DIM), jnp.float32),
        mesh=mesh,
        scratch_types=[
            pltpu.VMEM((PER_WORKER,), jnp.int32),
            pltpu.VMEM((NBUF, CHUNK, EMBEDDING_DIM), jnp.float32),
            [pltpu.SemaphoreType.DMA] * NBUF,
            [pltpu.SemaphoreType.DMA] * NBUF,
        ],
        compiler_params=pltpu.CompilerParams(use_tc_tiling_on_sc=False),
    )(flat_x, table)
    return out.reshape(BATCH, HIST, EMBEDDING_DIM)


# speedup vs baseline: 3.8864x; 2.0983x over previous
import jax
import jax.numpy as jnp
from jax import lax
from jax.experimental import pallas as pl
from jax.experimental.pallas import tpu as pltpu
from jax.experimental.pallas import tpu_sc as plsc

NUM_EMBEDDINGS = 1000000
EMBEDDING_DIM = 64
BATCH = 16384
HIST = 20

TOTAL = BATCH * HIST
NUM_CORES = 2
NUM_SUBCORES = 16
NUM_WORKERS = NUM_CORES * NUM_SUBCORES
PER_WORKER = TOTAL // NUM_WORKERS
CHUNK = 512
NUM_CHUNKS = PER_WORKER // CHUNK
NBUF = 3

LANES = 128
FULL_TC = NUM_EMBEDDINGS // LANES
TAIL = NUM_EMBEDDINGS - FULL_TC * LANES
TC_LO = FULL_TC // NUM_WORKERS
TC_EXTRA = FULL_TC - TC_LO * NUM_WORKERS

assert TOTAL % NUM_WORKERS == 0
assert PER_WORKER % CHUNK == 0


PITCH = 65


def _relayout_body(wt_hbm, tail_hbm, out_hbm, in0, in1, ob0, ob1, pad_buf,
                   isems, osems):
    in_bufs = (in0, in1)
    out_bufs = (ob0, ob1)
    wid = lax.axis_index("s") * NUM_CORES + lax.axis_index("c")
    base_tc = wid * TC_LO + jnp.minimum(wid, TC_EXTRA)
    n_tc = TC_LO + jnp.where(wid < TC_EXTRA, 1, 0)

    lane_iota = lax.iota(jnp.int32, 16)
    bases = [lane_iota * PITCH + 16 * k * PITCH
             for k in range(LANES // 16)]

    def start_load(tc, b):
        return pltpu.async_copy(
            wt_hbm.at[:, pl.ds(tc * LANES, LANES)], in_bufs[b], isems[b])

    def wait_load(b):
        pltpu.make_async_copy(
            wt_hbm.at[:, pl.ds(0, LANES)], in_bufs[b], isems[b]).wait()

    def start_store(tc, b):
        pltpu.async_copy(
            out_bufs[b],
            out_hbm.at[pl.ds(tc * LANES * EMBEDDING_DIM,
                             LANES * EMBEDDING_DIM)],
            osems[b])

    def wait_store(b):
        pltpu.make_async_copy(
            out_bufs[b],
            out_hbm.at[pl.ds(0, LANES * EMBEDDING_DIM)], osems[b]).wait()

    def transpose_block(b):
        inb = in_bufs[b]
        outb = out_bufs[b]

        @plsc.parallel_loop(0, EMBEDDING_DIM, step=1)
        def _(d):
            rows = jnp.full((16,), d, jnp.int32)
            for k in range(LANES // 16):
                v = plsc.load_gather(inb, [rows, lane_iota + 16 * k])
                plsc.store_scatter(pad_buf, [bases[k] + d], v)

        @plsc.parallel_loop(0, LANES, step=1)
        def _(vl):
            for j in range(EMBEDDING_DIM // 16):
                outb[pl.ds(vl * EMBEDDING_DIM + 16 * j, 16)] = (
                    pad_buf[pl.ds(vl * PITCH + 16 * j, 16)])

    start_load(base_tc, 0)
    start_load(base_tc + 1, 1)

    def pair(g, carry):
        for b in (0, 1):
            i = 2 * g + b

            @pl.when(i < n_tc)
            def _():
                tc = base_tc + i
                wait_load(b)

                @pl.when(i >= 2)
                def _():
                    wait_store(b)

                transpose_block(b)
                start_store(tc, b)

                @pl.when(i + 2 < n_tc)
                def _():
                    start_load(tc + 2, b)
        return carry

    lax.fori_loop(0, (TC_LO + 2) // 2, pair, 0)
    wait_store(0)
    wait_store(1)

    @pl.when(wid == NUM_WORKERS - 1)
    def _tail():
        pltpu.async_copy(
            tail_hbm, out_bufs[1].at[pl.ds(0, TAIL * EMBEDDING_DIM)],
            isems[1]).wait()
        pltpu.async_copy(
            out_bufs[1].at[pl.ds(0, TAIL * EMBEDDING_DIM)],
            out_hbm.at[pl.ds(FULL_TC * LANES * EMBEDDING_DIM,
                             TAIL * EMBEDDING_DIM)],
            osems[1]).wait()


def _gather_body(x_hbm, w_hbm, out_hbm, idx_v, rows_v, gsems, ssems):
    wid = lax.axis_index("s") * NUM_CORES + lax.axis_index("c")
    base = wid * PER_WORKER

    pltpu.sync_copy(x_hbm.at[pl.ds(base, PER_WORKER)], idx_v)

    def start_gather(c):
        b = c % NBUF
        return pltpu.async_copy(
            w_hbm.at[idx_v.at[pl.ds(c * CHUNK, CHUNK)]], rows_v.at[b],
            gsems[b])

    gd = {}
    for c in range(NBUF):
        gd[c] = start_gather(c)

    pending_stores = {}
    for c in range(NUM_CHUNKS):
        b = c % NBUF
        gd[c].wait()
        sd = pltpu.async_copy(
            rows_v.at[b], out_hbm.at[pl.ds(base + c * CHUNK, CHUNK)],
            ssems[b])
        if c + NBUF < NUM_CHUNKS:
            sd.wait()
            gd[c + NBUF] = start_gather(c + NBUF)
        else:
            pending_stores[b] = sd

    for b in sorted(pending_stores):
        pending_stores[b].wait()


@jax.jit
def kernel(x, weight):
    mesh = plsc.VectorSubcoreMesh(
        core_axis_name="c", subcore_axis_name="s",
        num_cores=NUM_CORES, num_subcores=NUM_SUBCORES,
    )
    dense = pl.kernel(
        _relayout_body,
        out_type=jax.ShapeDtypeStruct((NUM_EMBEDDINGS * EMBEDDING_DIM,),
                                      jnp.float32),
        mesh=mesh,
        scratch_types=[
            pltpu.VMEM((EMBEDDING_DIM, LANES), jnp.float32),
            pltpu.VMEM((EMBEDDING_DIM, LANES), jnp.float32),
            pltpu.VMEM((LANES * EMBEDDING_DIM,), jnp.float32),
            pltpu.VMEM((LANES * EMBEDDING_DIM,), jnp.float32),
            pltpu.VMEM((LANES * PITCH,), jnp.float32),
            [pltpu.SemaphoreType.DMA] * 2,
            [pltpu.SemaphoreType.DMA] * 2,
        ],
        compiler_params=pltpu.CompilerParams(use_tc_tiling_on_sc=True,
                                             needs_layout_passes=False),
    )(weight.T,
      lax.slice(weight, (FULL_TC * LANES, 0),
                (NUM_EMBEDDINGS, EMBEDDING_DIM)).reshape(TAIL * EMBEDDING_DIM))
    table = dense.reshape(NUM_EMBEDDINGS, EMBEDDING_DIM)

    flat_x = x.reshape(TOTAL)
    out = pl.kernel(
        _gather_body,
        out_type=jax.ShapeDtypeStruct((TOTAL, EMBEDDING_DIM), jnp.float32),
        mesh=mesh,
        scratch_types=[
            pltpu.VMEM((PER_WORKER,), jnp.int32),
            pltpu.VMEM((NBUF, CHUNK, EMBEDDING_DIM), jnp.float32),
            [pltpu.SemaphoreType.DMA] * NBUF,
            [pltpu.SemaphoreType.DMA] * NBUF,
        ],
        compiler_params=pltpu.CompilerParams(use_tc_tiling_on_sc=False),
    )(flat_x, table)
    return out.reshape(BATCH, HIST, EMBEDDING_DIM)


# speedup vs baseline: 4.4888x; 1.1550x over previous
import jax
import jax.numpy as jnp
from jax import lax
from jax.experimental import pallas as pl
from jax.experimental.pallas import tpu as pltpu
from jax.experimental.pallas import tpu_sc as plsc

NUM_EMBEDDINGS = 1000000
EMBEDDING_DIM = 64
BATCH = 16384
HIST = 20

TOTAL = BATCH * HIST
NUM_CORES = 2
NUM_SUBCORES = 16
NUM_WORKERS = NUM_CORES * NUM_SUBCORES
PER_WORKER = TOTAL // NUM_WORKERS
CHUNK = 512
NUM_CHUNKS = PER_WORKER // CHUNK
NBUF = 3

LANES = 128
FULL_TC = NUM_EMBEDDINGS // LANES
TAIL = NUM_EMBEDDINGS - FULL_TC * LANES
TC_LO = FULL_TC // NUM_WORKERS
TC_EXTRA = FULL_TC - TC_LO * NUM_WORKERS

assert TOTAL % NUM_WORKERS == 0
assert PER_WORKER % CHUNK == 0


PITCH = 65


def _relayout_body(wt_hbm, tail_hbm, out_hbm, in0, in1, ob0, ob1, pad_buf,
                   isems, osems):
    in_bufs = (in0, in1)
    out_bufs = (ob0, ob1)
    wid = lax.axis_index("s") * NUM_CORES + lax.axis_index("c")
    base_tc = wid * TC_LO + jnp.minimum(wid, TC_EXTRA)
    n_tc = TC_LO + jnp.where(wid < TC_EXTRA, 1, 0)

    lane_iota = lax.iota(jnp.int32, 16)
    bases = [lane_iota * PITCH + 16 * k * PITCH
             for k in range(LANES // 16)]

    def start_load(tc, b):
        return pltpu.async_copy(
            wt_hbm.at[:, pl.ds(tc * LANES, LANES)], in_bufs[b], isems[b])

    def wait_load(b):
        pltpu.make_async_copy(
            wt_hbm.at[:, pl.ds(0, LANES)], in_bufs[b], isems[b]).wait()

    def start_store(tc, b):
        pltpu.async_copy(
            out_bufs[b],
            out_hbm.at[pl.ds(tc * LANES * EMBEDDING_DIM,
                             LANES * EMBEDDING_DIM)],
            osems[b])

    def wait_store(b):
        pltpu.make_async_copy(
            out_bufs[b],
            out_hbm.at[pl.ds(0, LANES * EMBEDDING_DIM)], osems[b]).wait()

    def transpose_block(b):
        inb = in_bufs[b]
        outb = out_bufs[b]

        @plsc.parallel_loop(0, EMBEDDING_DIM, step=1)
        def _(d):
            rows = jnp.full((16,), d, jnp.int32)
            for k in range(LANES // 16):
                v = plsc.load_gather(inb, [rows, lane_iota + 16 * k])
                plsc.store_scatter(pad_buf, [bases[k] + d], v)

        @plsc.parallel_loop(0, LANES, step=1)
        def _(vl):
            for j in range(EMBEDDING_DIM // 16):
                outb[pl.ds(vl * EMBEDDING_DIM + 16 * j, 16)] = (
                    pad_buf[pl.ds(vl * PITCH + 16 * j, 16)])

    start_load(base_tc, 0)
    start_load(base_tc + 1, 1)

    def pair(g, carry):
        for b in (0, 1):
            i = 2 * g + b

            @pl.when(i < n_tc)
            def _():
                tc = base_tc + i
                wait_load(b)

                @pl.when(i >= 2)
                def _():
                    wait_store(b)

                transpose_block(b)
                start_store(tc, b)

                @pl.when(i + 2 < n_tc)
                def _():
                    start_load(tc + 2, b)
        return carry

    lax.fori_loop(0, (TC_LO + 2) // 2, pair, 0)
    wait_store(0)
    wait_store(1)

    @pl.when(wid == NUM_WORKERS - 1)
    def _tail():
        pltpu.async_copy(
            tail_hbm, out_bufs[1].at[pl.ds(0, TAIL * EMBEDDING_DIM)],
            isems[1]).wait()
        pltpu.async_copy(
            out_bufs[1].at[pl.ds(0, TAIL * EMBEDDING_DIM)],
            out_hbm.at[pl.ds(FULL_TC * LANES * EMBEDDING_DIM,
                             TAIL * EMBEDDING_DIM)],
            osems[1]).wait()


def _gather_body(x_hbm, w_hbm, out_hbm, idx_v, rows_v, gsems, ssems):
    wid = lax.axis_index("s") * NUM_CORES + lax.axis_index("c")
    base = wid * PER_WORKER

    pltpu.sync_copy(x_hbm.at[pl.ds(base, PER_WORKER)], idx_v)

    def start_gather(c):
        b = c % NBUF
        return pltpu.async_copy(
            w_hbm.at[idx_v.at[pl.ds(c * CHUNK, CHUNK)]], rows_v.at[b],
            gsems[b])

    gd = {}
    for c in range(NBUF):
        gd[c] = start_gather(c)

    pending_stores = {}
    for c in range(NUM_CHUNKS):
        b = c % NBUF
        gd[c].wait()
        sd = pltpu.async_copy(
            rows_v.at[b], out_hbm.at[pl.ds(base + c * CHUNK, CHUNK)],
            ssems[b])
        if c + NBUF < NUM_CHUNKS:
            sd.wait()
            gd[c + NBUF] = start_gather(c + NBUF)
        else:
            pending_stores[b] = sd

    for b in sorted(pending_stores):
        pending_stores[b].wait()


C_HP = HIST // 2
C_BC = BATCH // LANES
C_BLOCKS = C_HP * C_BC
C_PER_W = C_BLOCKS // NUM_WORKERS
C_PITCH = 129


def _format_body(din_hbm, out_hbm, idxb, rows0, rows1, op0, op1, pitchb,
                 gsems, osems):
    rows_bufs = (rows0, rows1)
    out_bufs = (op0, op1)
    wid = lax.axis_index("s") * NUM_CORES + lax.axis_index("c")
    base = wid * C_PER_W
    lane_iota = lax.iota(jnp.int32, 16)
    zeros16 = jnp.zeros((16,), jnp.int32)

    def blk(t, carry):
        blkid = base + t
        hp = blkid // C_BC
        bc = blkid % C_BC
        b = 0
        for k in range(LANES // 16):
            idxb[pl.ds(16 * k, 16)] = (
                (lane_iota + (16 * k) + bc * LANES) * C_HP + hp)
        pltpu.async_copy(din_hbm.at[idxb], rows_bufs[b], gsems[b]).wait()

        @plsc.parallel_loop(0, LANES, step=1)
        def _(l):
            rows = jnp.full((16,), l, jnp.int32)
            for j in range(LANES // 16):
                v = plsc.load_gather(rows_bufs[b], [rows, lane_iota + 16 * j])
                pitchb[pl.ds(l * C_PITCH + 16 * j, 16)] = v

        for p in (0, 1):
            @plsc.parallel_loop(0, EMBEDDING_DIM, step=1)
            def _(d):
                for k in range(LANES // 16):
                    idxv = ((lane_iota + 16 * k) * C_PITCH
                            + p * EMBEDDING_DIM + d)
                    v = plsc.load_gather(pitchb, [idxv])
                    plsc.store_scatter(
                        out_bufs[p], [zeros16, jnp.full((16,), d, jnp.int32),
                                      lane_iota + 16 * k], v)
            pltpu.async_copy(
                out_bufs[p],
                out_hbm.at[pl.ds(2 * hp + p, 1), :, pl.ds(bc * LANES, LANES)],
                osems[p]).wait()
        return carry

    lax.fori_loop(0, C_PER_W, blk, 0)


@jax.jit
def kernel(x, weight):
    mesh = plsc.VectorSubcoreMesh(
        core_axis_name="c", subcore_axis_name="s",
        num_cores=NUM_CORES, num_subcores=NUM_SUBCORES,
    )
    dense = pl.kernel(
        _relayout_body,
        out_type=jax.ShapeDtypeStruct((NUM_EMBEDDINGS * EMBEDDING_DIM,),
                                      jnp.float32),
        mesh=mesh,
        scratch_types=[
            pltpu.VMEM((EMBEDDING_DIM, LANES), jnp.float32),
            pltpu.VMEM((EMBEDDING_DIM, LANES), jnp.float32),
            pltpu.VMEM((LANES * EMBEDDING_DIM,), jnp.float32),
            pltpu.VMEM((LANES * EMBEDDING_DIM,), jnp.float32),
            pltpu.VMEM((LANES * PITCH,), jnp.float32),
            [pltpu.SemaphoreType.DMA] * 2,
            [pltpu.SemaphoreType.DMA] * 2,
        ],
        compiler_params=pltpu.CompilerParams(use_tc_tiling_on_sc=True,
                                             needs_layout_passes=False),
    )(weight.T,
      lax.slice(weight, (FULL_TC * LANES, 0),
                (NUM_EMBEDDINGS, EMBEDDING_DIM)).reshape(TAIL * EMBEDDING_DIM))
    table = dense.reshape(NUM_EMBEDDINGS, EMBEDDING_DIM)

    flat_x = x.reshape(TOTAL)
    out = pl.kernel(
        _gather_body,
        out_type=jax.ShapeDtypeStruct((TOTAL, EMBEDDING_DIM), jnp.float32),
        mesh=mesh,
        scratch_types=[
            pltpu.VMEM((PER_WORKER,), jnp.int32),
            pltpu.VMEM((NBUF, CHUNK, EMBEDDING_DIM), jnp.float32),
            [pltpu.SemaphoreType.DMA] * NBUF,
            [pltpu.SemaphoreType.DMA] * NBUF,
        ],
        compiler_params=pltpu.CompilerParams(use_tc_tiling_on_sc=False),
    )(flat_x, table)

    out_t = pl.kernel(
        _format_body,
        out_type=jax.ShapeDtypeStruct((HIST, EMBEDDING_DIM, BATCH),
                                      jnp.float32),
        mesh=mesh,
        scratch_types=[
            pltpu.VMEM((LANES,), jnp.int32),
            pltpu.VMEM((LANES, LANES), jnp.float32),
            pltpu.VMEM((LANES, LANES), jnp.float32),
            pltpu.VMEM((1, EMBEDDING_DIM, LANES), jnp.float32),
            pltpu.VMEM((1, EMBEDDING_DIM, LANES), jnp.float32),
            pltpu.VMEM((LANES * C_PITCH,), jnp.float32),
            [pltpu.SemaphoreType.DMA] * 2,
            [pltpu.SemaphoreType.DMA] * 2,
        ],
        compiler_params=pltpu.CompilerParams(use_tc_tiling_on_sc=True,
                                             needs_layout_passes=False),
    )(out.reshape(TOTAL // 2, 2 * EMBEDDING_DIM))
    return jnp.transpose(out_t, (2, 0, 1))


# speedup vs baseline: 5.0968x; 1.1355x over previous
import jax
import jax.numpy as jnp
from jax import lax
from jax.experimental import pallas as pl
from jax.experimental.pallas import tpu as pltpu
from jax.experimental.pallas import tpu_sc as plsc

NUM_EMBEDDINGS = 1000000
EMBEDDING_DIM = 64
BATCH = 16384
HIST = 20

TOTAL = BATCH * HIST
NUM_CORES = 2
NUM_SUBCORES = 16
NUM_WORKERS = NUM_CORES * NUM_SUBCORES
PER_WORKER = TOTAL // NUM_WORKERS
CHUNK = 512
NUM_CHUNKS = PER_WORKER // CHUNK
NBUF = 3

LANES = 128
FULL_TC = NUM_EMBEDDINGS // LANES
TAIL = NUM_EMBEDDINGS - FULL_TC * LANES
TC_LO = FULL_TC // NUM_WORKERS
TC_EXTRA = FULL_TC - TC_LO * NUM_WORKERS

assert TOTAL % NUM_WORKERS == 0
assert PER_WORKER % CHUNK == 0


PITCH = 65


def _relayout_body(wt_hbm, tail_hbm, out_hbm, in0, in1, ob0, ob1, pad_buf,
                   isems, osems):
    in_bufs = (in0, in1)
    out_bufs = (ob0, ob1)
    wid = lax.axis_index("s") * NUM_CORES + lax.axis_index("c")
    base_tc = wid * TC_LO + jnp.minimum(wid, TC_EXTRA)
    n_tc = TC_LO + jnp.where(wid < TC_EXTRA, 1, 0)

    lane_iota = lax.iota(jnp.int32, 16)
    bases = [lane_iota * PITCH + 16 * k * PITCH
             for k in range(LANES // 16)]

    def start_load(tc, b):
        return pltpu.async_copy(
            wt_hbm.at[:, pl.ds(tc * LANES, LANES)], in_bufs[b], isems[b])

    def wait_load(b):
        pltpu.make_async_copy(
            wt_hbm.at[:, pl.ds(0, LANES)], in_bufs[b], isems[b]).wait()

    def start_store(tc, b):
        pltpu.async_copy(
            out_bufs[b],
            out_hbm.at[pl.ds(tc * LANES * EMBEDDING_DIM,
                             LANES * EMBEDDING_DIM)],
            osems[b])

    def wait_store(b):
        pltpu.make_async_copy(
            out_bufs[b],
            out_hbm.at[pl.ds(0, LANES * EMBEDDING_DIM)], osems[b]).wait()

    def transpose_block(b):
        inb = in_bufs[b]
        outb = out_bufs[b]

        @plsc.parallel_loop(0, EMBEDDING_DIM, step=1)
        def _(d):
            rows = jnp.full((16,), d, jnp.int32)
            for k in range(LANES // 16):
                v = plsc.load_gather(inb, [rows, lane_iota + 16 * k])
                plsc.store_scatter(pad_buf, [bases[k] + d], v)

        @plsc.parallel_loop(0, LANES, step=1)
        def _(vl):
            for j in range(EMBEDDING_DIM // 16):
                outb[pl.ds(vl * EMBEDDING_DIM + 16 * j, 16)] = (
                    pad_buf[pl.ds(vl * PITCH + 16 * j, 16)])

    start_load(base_tc, 0)
    start_load(base_tc + 1, 1)

    def pair(g, carry):
        for b in (0, 1):
            i = 2 * g + b

            @pl.when(i < n_tc)
            def _():
                tc = base_tc + i
                wait_load(b)

                @pl.when(i >= 2)
                def _():
                    wait_store(b)

                transpose_block(b)
                start_store(tc, b)

                @pl.when(i + 2 < n_tc)
                def _():
                    start_load(tc + 2, b)
        return carry

    lax.fori_loop(0, (TC_LO + 2) // 2, pair, 0)
    wait_store(0)
    wait_store(1)

    @pl.when(wid == NUM_WORKERS - 1)
    def _tail():
        pltpu.async_copy(
            tail_hbm, out_bufs[1].at[pl.ds(0, TAIL * EMBEDDING_DIM)],
            isems[1]).wait()
        pltpu.async_copy(
            out_bufs[1].at[pl.ds(0, TAIL * EMBEDDING_DIM)],
            out_hbm.at[pl.ds(FULL_TC * LANES * EMBEDDING_DIM,
                             TAIL * EMBEDDING_DIM)],
            osems[1]).wait()


def _gather_body(x_hbm, w_hbm, out_hbm, idx_v, rows_v, gsems, ssems):
    wid = lax.axis_index("s") * NUM_CORES + lax.axis_index("c")
    base = wid * PER_WORKER

    pltpu.sync_copy(x_hbm.at[pl.ds(base, PER_WORKER)], idx_v)

    def start_gather(c):
        b = c % NBUF
        return pltpu.async_copy(
            w_hbm.at[idx_v.at[pl.ds(c * CHUNK, CHUNK)]], rows_v.at[b],
            gsems[b])

    gd = {}
    for c in range(NBUF):
        gd[c] = start_gather(c)

    pending_stores = {}
    for c in range(NUM_CHUNKS):
        b = c % NBUF
        gd[c].wait()
        sd = pltpu.async_copy(
            rows_v.at[b], out_hbm.at[pl.ds(base + c * CHUNK, CHUNK)],
            ssems[b])
        if c + NBUF < NUM_CHUNKS:
            sd.wait()
            gd[c + NBUF] = start_gather(c + NBUF)
        else:
            pending_stores[b] = sd

    for b in sorted(pending_stores):
        pending_stores[b].wait()


C_HP = HIST // 2
C_BC = BATCH // LANES
C_BLOCKS = C_HP * C_BC
C_PER_W = C_BLOCKS // NUM_WORKERS
C_PITCH = 129


def _format_body(din_hbm, out_hbm, idxb, rows0, rows1, op0, op1, pitchb,
                 gsems, osems):
    rows_bufs = (rows0, rows1)
    out_bufs = (op0, op1)
    wid = lax.axis_index("s") * NUM_CORES + lax.axis_index("c")
    base = wid * C_PER_W
    lane_iota = lax.iota(jnp.int32, 16)
    zeros16 = jnp.zeros((16,), jnp.int32)

    def start_gather(t, b):
        blkid = base + t
        hp = blkid // C_BC
        bc = blkid % C_BC
        for k in range(LANES // 16):
            idxb[pl.ds(16 * k, 16)] = (
                (lane_iota + (16 * k) + bc * LANES) * C_HP + hp)
        pltpu.async_copy(din_hbm.at[idxb], rows_bufs[b], gsems[b])

    def wait_gather(b):
        pltpu.make_async_copy(
            din_hbm.at[pl.ds(0, LANES)], rows_bufs[b], gsems[b]).wait()

    start_gather(0, 0)

    def blk(t, carry):
        for b in (0, 1):
            @pl.when((lax.rem(t, 2) == b) & (t < C_PER_W))
            def _():
                blkid = base + t
                hp = blkid // C_BC
                bc = blkid % C_BC
                wait_gather(b)

                @pl.when(t + 1 < C_PER_W)
                def _():
                    start_gather(t + 1, 1 - b)

                @plsc.parallel_loop(0, LANES, step=1)
                def _(l):
                    rows = jnp.full((16,), l, jnp.int32)
                    for j in range(LANES // 16):
                        v = plsc.load_gather(rows_bufs[b],
                                             [rows, lane_iota + 16 * j])
                        pitchb[pl.ds(l * C_PITCH + 16 * j, 16)] = v

                for p in (0, 1):
                    @plsc.parallel_loop(0, EMBEDDING_DIM, step=1)
                    def _(d):
                        for k in range(LANES // 16):
                            idxv = ((lane_iota + 16 * k) * C_PITCH
                                    + p * EMBEDDING_DIM + d)
                            v = plsc.load_gather(pitchb, [idxv])
                            plsc.store_scatter(
                                out_bufs[p],
                                [zeros16, jnp.full((16,), d, jnp.int32),
                                 lane_iota + 16 * k], v)
                    pltpu.async_copy(
                        out_bufs[p],
                        out_hbm.at[pl.ds(2 * hp + p, 1), :,
                                   pl.ds(bc * LANES, LANES)],
                        osems[p]).wait()
        return carry

    lax.fori_loop(0, C_PER_W, blk, 0)


@jax.jit
def kernel(x, weight):
    mesh = plsc.VectorSubcoreMesh(
        core_axis_name="c", subcore_axis_name="s",
        num_cores=NUM_CORES, num_subcores=NUM_SUBCORES,
    )
    dense = pl.kernel(
        _relayout_body,
        out_type=jax.ShapeDtypeStruct((NUM_EMBEDDINGS * EMBEDDING_DIM,),
                                      jnp.float32),
        mesh=mesh,
        scratch_types=[
            pltpu.VMEM((EMBEDDING_DIM, LANES), jnp.float32),
            pltpu.VMEM((EMBEDDING_DIM, LANES), jnp.float32),
            pltpu.VMEM((LANES * EMBEDDING_DIM,), jnp.float32),
            pltpu.VMEM((LANES * EMBEDDING_DIM,), jnp.float32),
            pltpu.VMEM((LANES * PITCH,), jnp.float32),
            [pltpu.SemaphoreType.DMA] * 2,
            [pltpu.SemaphoreType.DMA] * 2,
        ],
        compiler_params=pltpu.CompilerParams(use_tc_tiling_on_sc=True,
                                             needs_layout_passes=False),
    )(weight.T,
      lax.slice(weight, (FULL_TC * LANES, 0),
                (NUM_EMBEDDINGS, EMBEDDING_DIM)).reshape(TAIL * EMBEDDING_DIM))
    table = dense.reshape(NUM_EMBEDDINGS, EMBEDDING_DIM)

    flat_x = x.reshape(TOTAL)
    out = pl.kernel(
        _gather_body,
        out_type=jax.ShapeDtypeStruct((TOTAL, EMBEDDING_DIM), jnp.float32),
        mesh=mesh,
        scratch_types=[
            pltpu.VMEM((PER_WORKER,), jnp.int32),
            pltpu.VMEM((NBUF, CHUNK, EMBEDDING_DIM), jnp.float32),
            [pltpu.SemaphoreType.DMA] * NBUF,
            [pltpu.SemaphoreType.DMA] * NBUF,
        ],
        compiler_params=pltpu.CompilerParams(use_tc_tiling_on_sc=False),
    )(flat_x, table)

    out_t = pl.kernel(
        _format_body,
        out_type=jax.ShapeDtypeStruct((HIST, EMBEDDING_DIM, BATCH),
                                      jnp.float32),
        mesh=mesh,
        scratch_types=[
            pltpu.VMEM((LANES,), jnp.int32),
            pltpu.VMEM((LANES, LANES), jnp.float32),
            pltpu.VMEM((LANES, LANES), jnp.float32),
            pltpu.VMEM((1, EMBEDDING_DIM, LANES), jnp.float32),
            pltpu.VMEM((1, EMBEDDING_DIM, LANES), jnp.float32),
            pltpu.VMEM((LANES * C_PITCH,), jnp.float32),
            [pltpu.SemaphoreType.DMA] * 2,
            [pltpu.SemaphoreType.DMA] * 2,
        ],
        compiler_params=pltpu.CompilerParams(use_tc_tiling_on_sc=True,
                                             needs_layout_passes=False),
    )(out.reshape(TOTAL // 2, 2 * EMBEDDING_DIM))
    return jnp.transpose(out_t, (2, 0, 1))
